# baseline (speedup 1.0000x reference)
import functools

import jax
import jax.numpy as jnp
from jax import lax
from jax.experimental import pallas as pl
from jax.experimental.pallas import tpu as pltpu
from jax.experimental.pallas import tpu_sc as plsc

_B = 4096
_S = 50
_NF = 26
_CARD = 100000
_EDIM = 32
_DMODEL = 128

_T = _B * _S
_R = _T * _NF
_LPB = 128
_CHUNK = 10

_BK = 33344
_QR = _BK // 4
_NB = 3
_PLINES = _NF * _NB * _QR


def _tc_repack(tables):
    def rk(a_ref, o_ref):
        a = a_ref[0]
        o_ref[...] = jnp.concatenate(
            [a[0:_QR], a[_QR : 2 * _QR], a[2 * _QR : 3 * _QR], a[3 * _QR :]],
            axis=1,
        )

    return pl.pallas_call(
        rk,
        grid=(_NF * _NB,),
        in_specs=[
            pl.BlockSpec((1, _BK, _EDIM), lambda g: (g // _NB, g % _NB, 0)),
        ],
        out_specs=pl.BlockSpec((_QR, _LPB), lambda g: (g, 0)),
        out_shape=jax.ShapeDtypeStruct((_PLINES, _LPB), jnp.float32),
    )(tables)


def _sc_gather(ptab32, gidx):
    info = plsc.get_sparse_core_info()
    nw = info.num_cores * info.num_subcores
    nblk = _R // _LPB
    blocks_per_w = nblk // nw
    iters = blocks_per_w // _CHUNK
    crows = _CHUNK * _LPB

    mesh = plsc.VectorSubcoreMesh(core_axis_name="c", subcore_axis_name="s")

    @functools.partial(
        pl.kernel,
        mesh=mesh,
        compiler_params=pltpu.CompilerParams(use_tc_tiling_on_sc=False),
        out_type=jax.ShapeDtypeStruct((_R, _EDIM), jnp.float32),
        scratch_types=[
            pltpu.VMEM((_CHUNK, _LPB), jnp.int32),
            pltpu.VMEM((_CHUNK, _LPB), jnp.int32),
            pltpu.VMEM((crows, _EDIM), jnp.float32),
            pltpu.VMEM((crows, _EDIM), jnp.float32),
            pltpu.SemaphoreType.DMA,
            pltpu.SemaphoreType.DMA,
        ],
    )
    def k(tab32, gidx_hbm, out_hbm, idx0, idx1, rows0, rows1, sem0, sem1):
        wid = lax.axis_index("s") * info.num_cores + lax.axis_index("c")
        base = wid * blocks_per_w

        def fire(chunk, idx_v, rows_v, sem):
            blk = base + chunk * _CHUNK
            pltpu.sync_copy(gidx_hbm.at[pl.ds(blk, _CHUNK)], idx_v)
            for j in range(_CHUNK):
                pltpu.async_copy(
                    tab32.at[idx_v.at[j]],
                    rows_v.at[pl.ds(j * _LPB, _LPB)],
                    sem,
                )

        def drain(idx_v, rows_v, sem):
            for j in range(_CHUNK):
                pltpu.make_async_copy(
                    tab32.at[idx_v.at[j]],
                    rows_v.at[pl.ds(j * _LPB, _LPB)],
                    sem,
                ).wait()

        def writeback(chunk, rows_v):
            blk = base + chunk * _CHUNK
            pltpu.sync_copy(rows_v, out_hbm.at[pl.ds(blk * _LPB, crows)])

        fire(0, idx0, rows0, sem0)

        def body(g, carry):
            c0 = 2 * g
            fire(c0 + 1, idx1, rows1, sem1)
            drain(idx0, rows0, sem0)
            writeback(c0, rows0)

            @pl.when(g < iters // 2 - 1)
            def _():
                fire(c0 + 2, idx0, rows0, sem0)

            drain(idx1, rows1, sem1)
            writeback(c0 + 1, rows1)
            return carry

        lax.fori_loop(0, iters // 2, body, 0)

    return k(ptab32, gidx)


def _tc_project(c3, m3, bias4):
    l4 = 512
    nt4 = _T // 4 // l4

    def mm(c_ref, m_ref, b_ref, o_ref):
        acc = b_ref[...].astype(jnp.float32) + jnp.zeros(
            (l4, 4 * _DMODEL), jnp.float32
        )
        for f in range(_NF):
            acc += jnp.dot(
                c_ref[f].astype(jnp.bfloat16),
                m_ref[f],
                preferred_element_type=jnp.float32,
            )
        o_ref[...] = acc

    return pl.pallas_call(
        mm,
        grid=(nt4,),
        in_specs=[
            pl.BlockSpec((_NF, l4, _LPB), lambda i: (0, i, 0)),
            pl.BlockSpec((_NF, _LPB, 4 * _DMODEL), lambda i: (0, 0, 0)),
            pl.BlockSpec((1, 4 * _DMODEL), lambda i: (0, 0)),
        ],
        out_specs=pl.BlockSpec((l4, 4 * _DMODEL), lambda i: (i, 0)),
        out_shape=jax.ShapeDtypeStruct((_T // 4, 4 * _DMODEL), jnp.float32),
    )(c3, m3, bias4)


def kernel(x_cat, tables, W, b):
    ptab = _tc_repack(tables)

    x3 = x_cat.astype(jnp.int32)
    foff = (jnp.arange(_NF, dtype=jnp.int32) * _NB)[None, None, :]
    rb = x3 // _BK
    ip = x3 % _BK
    j3 = (foff + rb) * _BK + 4 * (ip % _QR) + ip // _QR
    gidx = j3.transpose(2, 0, 1).reshape(_R // _LPB, _LPB)

    cat = _sc_gather(ptab.reshape(4 * _PLINES, _EDIM), gidx)
    c3 = cat.reshape(_NF, _T // 4, _LPB)

    w3 = W.reshape(_NF, _EDIM, _DMODEL).astype(jnp.bfloat16)
    eye4 = jnp.eye(4, dtype=jnp.bfloat16)
    m3 = jnp.einsum("fed,cq->fceqd", w3, eye4).reshape(
        _NF, _LPB, 4 * _DMODEL
    )
    bias4 = jnp.tile(b, 4).reshape(1, 4 * _DMODEL)

    out4 = _tc_project(c3, m3, bias4)
    return out4.reshape(_B, _S, _DMODEL)


# speedup vs baseline: 1.0143x; 1.0143x over previous
import functools

import jax
import jax.numpy as jnp
from jax import lax
from jax.experimental import pallas as pl
from jax.experimental.pallas import tpu as pltpu
from jax.experimental.pallas import tpu_sc as plsc

_B = 4096
_S = 50
_NF = 26
_NFH = 13
_CARD = 100000
_EDIM = 32
_DMODEL = 128

_T = _B * _S
_R = _T * _NF
_RH = _R // 2
_LPB = 128
_CHUNK = 13

_BK = 33344
_QR = _BK // 4
_NB = 3
_PLINESH = _NFH * _NB * _QR


def _tc_repack(tables, f0):
    def rk(a_ref, o_ref):
        a = a_ref[0]
        o_ref[...] = jnp.concatenate(
            [a[0:_QR], a[_QR : 2 * _QR], a[2 * _QR : 3 * _QR], a[3 * _QR :]],
            axis=1,
        )

    return pl.pallas_call(
        rk,
        grid=(_NFH * _NB,),
        in_specs=[
            pl.BlockSpec(
                (1, _BK, _EDIM), lambda g: (f0 + g // _NB, g % _NB, 0)
            ),
        ],
        out_specs=pl.BlockSpec((_QR, _LPB), lambda g: (g, 0)),
        out_shape=jax.ShapeDtypeStruct((_PLINESH, _LPB), jnp.float32),
    )(tables)


def _sc_gather(ptab32, gidx, half):
    info = plsc.get_sparse_core_info()
    nw = info.num_cores * info.num_subcores
    nblk = _RH // _LPB
    blocks_per_w = nblk // nw
    iters = blocks_per_w // _CHUNK
    crows = _CHUNK * _LPB

    mesh = plsc.VectorSubcoreMesh(core_axis_name="c", subcore_axis_name="s")

    @functools.partial(
        pl.kernel,
        mesh=mesh,
        compiler_params=pltpu.CompilerParams(use_tc_tiling_on_sc=False),
        out_type=jax.ShapeDtypeStruct((_RH, _EDIM), jnp.float32),
        scratch_types=[
            pltpu.VMEM((_CHUNK, _LPB), jnp.int32),
            pltpu.VMEM((_CHUNK, _LPB), jnp.int32),
            pltpu.VMEM((crows, _EDIM), jnp.float32),
            pltpu.VMEM((crows, _EDIM), jnp.float32),
            pltpu.SemaphoreType.DMA,
            pltpu.SemaphoreType.DMA,
        ],
    )
    def k(tab32, gidx_hbm, out_hbm, idx0, idx1, rows0, rows1, sem0, sem1):
        wid = lax.axis_index("s") * info.num_cores + lax.axis_index("c")
        base = half * nblk + wid * blocks_per_w
        obase = wid * blocks_per_w

        def fire(chunk, idx_v, rows_v, sem):
            blk = base + chunk * _CHUNK
            pltpu.sync_copy(gidx_hbm.at[pl.ds(blk, _CHUNK)], idx_v)
            for j in range(_CHUNK):
                pltpu.async_copy(
                    tab32.at[idx_v.at[j]],
                    rows_v.at[pl.ds(j * _LPB, _LPB)],
                    sem,
                )

        def drain(idx_v, rows_v, sem):
            for j in range(_CHUNK):
                pltpu.make_async_copy(
                    tab32.at[idx_v.at[j]],
                    rows_v.at[pl.ds(j * _LPB, _LPB)],
                    sem,
                ).wait()

        def writeback(chunk, rows_v):
            blk = obase + chunk * _CHUNK
            pltpu.sync_copy(rows_v, out_hbm.at[pl.ds(blk * _LPB, crows)])

        fire(0, idx0, rows0, sem0)

        def body(g, carry):
            c0 = 2 * g
            fire(c0 + 1, idx1, rows1, sem1)
            drain(idx0, rows0, sem0)
            writeback(c0, rows0)

            @pl.when(g < iters // 2 - 1)
            def _():
                fire(c0 + 2, idx0, rows0, sem0)

            drain(idx1, rows1, sem1)
            writeback(c0 + 1, rows1)
            return carry

        lax.fori_loop(0, iters // 2, body, 0)

    return k(ptab32, gidx)


def _tc_project(c3a, c3b, m3, bias4):
    l4 = 512
    nt4 = _T // 4 // l4

    def mm(ca_ref, cb_ref, m_ref, b_ref, o_ref):
        acc = b_ref[...].astype(jnp.float32) + jnp.zeros(
            (l4, 4 * _DMODEL), jnp.float32
        )
        for f in range(_NFH):
            acc += jnp.dot(
                ca_ref[f].astype(jnp.bfloat16),
                m_ref[f],
                preferred_element_type=jnp.float32,
            )
        for f in range(_NFH):
            acc += jnp.dot(
                cb_ref[f].astype(jnp.bfloat16),
                m_ref[_NFH + f],
                preferred_element_type=jnp.float32,
            )
        o_ref[...] = acc

    return pl.pallas_call(
        mm,
        grid=(nt4,),
        in_specs=[
            pl.BlockSpec((_NFH, l4, _LPB), lambda i: (0, i, 0)),
            pl.BlockSpec((_NFH, l4, _LPB), lambda i: (0, i, 0)),
            pl.BlockSpec((_NF, _LPB, 4 * _DMODEL), lambda i: (0, 0, 0)),
            pl.BlockSpec((1, 4 * _DMODEL), lambda i: (0, 0)),
        ],
        out_specs=pl.BlockSpec((l4, 4 * _DMODEL), lambda i: (i, 0)),
        out_shape=jax.ShapeDtypeStruct((_T // 4, 4 * _DMODEL), jnp.float32),
    )(c3a, c3b, m3, bias4)


def kernel(x_cat, tables, W, b):
    x3 = x_cat.astype(jnp.int32)
    foff = ((jnp.arange(_NF, dtype=jnp.int32) % _NFH) * _NB)[None, None, :]
    rb = x3 // _BK
    ip = x3 % _BK
    j3 = (foff + rb) * _BK + 4 * (ip % _QR) + ip // _QR
    gidx = j3.transpose(2, 0, 1).reshape(_R // _LPB, _LPB)

    ptab_a = _tc_repack(tables, 0)
    cat_a = _sc_gather(ptab_a.reshape(4 * _PLINESH, _EDIM), gidx, 0)
    ptab_b = _tc_repack(tables, _NFH)
    cat_b = _sc_gather(ptab_b.reshape(4 * _PLINESH, _EDIM), gidx, 1)

    c3a = cat_a.reshape(_NFH, _T // 4, _LPB)
    c3b = cat_b.reshape(_NFH, _T // 4, _LPB)

    w3 = W.reshape(_NF, _EDIM, _DMODEL).astype(jnp.bfloat16)
    eye4 = jnp.eye(4, dtype=jnp.bfloat16)
    m3 = jnp.einsum("fed,cq->fceqd", w3, eye4).reshape(
        _NF, _LPB, 4 * _DMODEL
    )
    bias4 = jnp.tile(b, 4).reshape(1, 4 * _DMODEL)

    out4 = _tc_project(c3a, c3b, m3, bias4)
    return out4.reshape(_B, _S, _DMODEL)
